# Initial kernel scaffold; baseline (speedup 1.0000x reference)
import jax
import jax.numpy as jnp
from jax.experimental import pallas as pl


def kernel(x, enc_W1, enc_b1, enc_g, enc_bln, enc_W2, enc_b2, dec_W1, dec_b1, dec_g, dec_bln, dec_W2, dec_b2, pattern_memory, pattern_importance):
    raise NotImplementedError("write your pallas kernel here")


# speedup vs baseline: 2.5487x; 2.5487x over previous
import functools

import jax
import jax.numpy as jnp
from jax.experimental import pallas as pl

HIDDEN = 768
NUM_PATTERNS = 256
TOP_K = 8
CHUNK = 128
EPS = 1e-5


def _atanh(x):
    return 0.5 * (jnp.log1p(x) - jnp.log1p(-x))


def _erfc(z):
    f = jnp.float32
    a = jnp.abs(z)
    w = z * z
    p = w * f(7.85386146e-05) + f(-0.000801019371)
    p = p * w + f(0.00518832775)
    p = p * w + f(-0.0268538129)
    p = p * w + f(0.112835854)
    p = p * w + f(-0.37612626)
    p = p * w + f(1.12837911)
    branch_a = f(1.0) - z * p
    nw = -w
    ez = jnp.exp(nw)
    ezr = ez * (f(1.0) / a)
    s = f(1.0) / w
    q = s * f(0.0232682) + f(-0.138703942)
    q = q * s + f(0.368742466)
    q = q * s + f(-0.582473278)
    q = q * s + f(0.621000469)
    q = q * s + f(-0.494451523)
    q = q * s + f(0.340488)
    q = q * s + f(-0.274112701)
    q = q * s + f(0.563825965)
    r = s * f(-10.477664) + f(12.9772)
    r = r * s + f(-7.49551868)
    r = r * s + f(2.92101908)
    r = r * s + f(-1.01526523)
    r = r * s + f(0.42184633)
    r = r * s + f(-0.282076746)
    r = r * s + f(0.564189494)
    y = ezr * jnp.where(a < f(2.0), q, r)
    y = jnp.where(nw < f(-88.7228394), f(0.0), y)
    branch_b = jnp.where(z < f(0.0), f(2.0) - y, y)
    return jnp.where(a < f(1.0), branch_a, branch_b)


def _acosh(x):
    return jnp.log(x + jnp.sqrt((x - 1.0) * (x + 1.0)))


def _sum768(v):
    acc = v[:, 0:128]
    for i in range(1, 6):
        acc = acc + v[:, i * 128:(i + 1) * 128]
    r = acc[:, 0:8]
    for i in range(1, 16):
        r = r + acc[:, i * 8:(i + 1) * 8]
    t = r[:, 0:4] + r[:, 4:8]
    t = t[:, 0:2] + t[:, 2:4]
    return t[:, 0:1] + t[:, 1:2]


def _rowsum(v):
    if v.shape[-1] == 768:
        return _sum768(v)
    return jnp.sum(v, axis=-1, keepdims=True)


def _log0(x):
    n = jnp.sqrt(_rowsum(x * x))
    n_c = jnp.clip(n, 1e-9, 1.0 - EPS)
    return _atanh(n_c) * x / jnp.maximum(n, 1e-9)


def _exp0(v):
    n = jnp.sqrt(_rowsum(v * v))
    return jnp.tanh(n) * v / jnp.maximum(n, 1e-9)


def _mlp(t, W1, b1, g, bln, W2, b2):
    h = jnp.dot(t, W1, preferred_element_type=jnp.float32) + b1
    mu = jnp.mean(h, axis=-1, keepdims=True)
    var = jnp.var(h, axis=-1, keepdims=True)
    h = (h - mu) / jnp.sqrt(var + 1e-5) * g + bln
    h = (h * 0.5) * _erfc(-h * jnp.float32(0.707106769))
    return jnp.dot(h, W2, preferred_element_type=jnp.float32) + b2


def _enc_body(x_ref, w1_ref, b1_ref, g_ref, bln_ref, w2_ref, b2_ref, out_ref):
    t = _log0(x_ref[...])
    y = _mlp(t, w1_ref[...], b1_ref[...], g_ref[...], bln_ref[...],
             w2_ref[...], b2_ref[...])
    out_ref[...] = _exp0(y)


def _ret_dec_body(pat_ref, pm_ref, pm128_ref, imp_ref,
                  w1_ref, b1_ref, g_ref, bln_ref, w2_ref, b2_ref, out_ref):
    c = pat_ref[...]
    pm = pm_ref[...]
    x2 = jnp.clip(_sum768(c * c)[:, 0], 0.0, 1.0 - EPS)
    y2 = jnp.clip(_sum768(pm * pm)[:, 0], 0.0, 1.0 - EPS)
    xyt = jax.lax.dot_general(pm, c, (((1,), (1,)), ((), ())),
                              preferred_element_type=jnp.float32)
    diff2 = jnp.maximum(y2[:, None] + x2[None, :] - 2.0 * xyt, 0.0)
    denom = jnp.maximum((1.0 - y2)[:, None] * (1.0 - x2)[None, :], 1e-12)
    arg = jnp.maximum(1.0 + 2.0 * diff2 / denom, 1.0 + 1e-7)
    dist = _acosh(arg)
    w = jax.nn.sigmoid(imp_ref[...])
    d = dist * w.reshape(NUM_PATTERNS, 1)

    col = jax.lax.broadcasted_iota(jnp.int32, d.shape, 1)
    simw = jnp.exp(-d)
    work = d
    sel_w = jnp.zeros_like(d)
    for _ in range(TOP_K):
        m = jnp.min(work, axis=-1, keepdims=True)
        ismin = work == m
        first = jnp.min(jnp.where(ismin, col, CHUNK), axis=-1, keepdims=True)
        sel = col == first
        sel_w = jnp.where(sel, simw, sel_w)
        work = jnp.where(sel, jnp.float32(1e30), work)

    weighted = jnp.dot(sel_w, pm128_ref[...],
                       preferred_element_type=jnp.float32)
    t = _log0(weighted)
    y = _mlp(t, w1_ref[...], b1_ref[...], g_ref[...], bln_ref[...],
             w2_ref[...], b2_ref[...])
    out_ref[...] = _exp0(y)


@functools.partial(jax.jit, static_argnames=("interpret",))
def kernel(x, enc_W1, enc_b1, enc_g, enc_bln, enc_W2, enc_b2,
           dec_W1, dec_b1, dec_g, dec_bln, dec_W2, dec_b2,
           pattern_memory, pattern_importance, interpret=False):
    B, S, D = x.shape
    ntok = B * S
    xf = x.reshape(ntok, D)
    blk = 256
    nblk = ntok // blk

    full = lambda shape: pl.BlockSpec(shape, lambda i: (0,) * len(shape))
    row2d = lambda r, c: pl.BlockSpec((r, c), lambda i: (i, 0))

    b1e = enc_b1.reshape(1, -1)
    ge = enc_g.reshape(1, -1)
    blne = enc_bln.reshape(1, -1)
    pattern = pl.pallas_call(
        _enc_body,
        grid=(nblk,),
        in_specs=[row2d(blk, D), full(enc_W1.shape), full(b1e.shape),
                  full(ge.shape), full(blne.shape), full(enc_W2.shape),
                  full((1, D))],
        out_specs=row2d(blk, D),
        out_shape=jax.ShapeDtypeStruct((ntok, D), jnp.float32),
        interpret=interpret,
    )(xf, enc_W1, b1e, ge, blne, enc_W2, enc_b2.reshape(1, -1))

    nchunk_active = ntok // NUM_PATTERNS
    pat_act = pattern[: nchunk_active * CHUNK]
    b1d = dec_b1.reshape(1, -1)
    gd = dec_g.reshape(1, -1)
    blnd = dec_bln.reshape(1, -1)
    imp = pattern_importance.reshape(1, NUM_PATTERNS)
    output = pl.pallas_call(
        _ret_dec_body,
        grid=(nchunk_active,),
        in_specs=[row2d(CHUNK, D), full(pattern_memory.shape),
                  full((CHUNK, D)), full(imp.shape),
                  full(dec_W1.shape), full(b1d.shape), full(gd.shape),
                  full(blnd.shape), full(dec_W2.shape), full((1, D))],
        out_specs=row2d(NUM_PATTERNS, D),
        out_shape=jax.ShapeDtypeStruct((ntok, D), jnp.float32),
        interpret=interpret,
    )(pat_act, pattern_memory, pattern_memory[:CHUNK], imp,
      dec_W1, b1d, gd, blnd, dec_W2, dec_b2.reshape(1, -1))

    return output.reshape(B, S, D), pattern.reshape(B, S, D)
